# baseline (speedup 1.0000x reference)
import functools

import jax
import jax.numpy as jnp
from jax import lax
from jax.experimental import pallas as pl
from jax.experimental.pallas import tpu as pltpu
from jax.experimental.pallas import tpu_sc as plsc

_NUM_CORES = 2
_NUM_SUBCORES = 16
_NW = _NUM_CORES * _NUM_SUBCORES
_NB = 2
_NBUF = 4
_NSPLIT = 2


def _gather_body(n_chunks, h, idx_hbm, w_hbm, out_hbm, idx_v, rows_v, sems):
    wid = lax.axis_index("s") * _NUM_CORES + lax.axis_index("c")
    pltpu.sync_copy(idx_hbm.at[wid], idx_v)
    batch0 = wid * (n_chunks * _NB)

    for b in range(_NBUF):
        pltpu.async_copy(w_hbm.at[idx_v.at[b]], rows_v.at[b], sems.at[b])

    def group(g, carry):
        for b in range(_NBUF):
            c = g * _NBUF + b
            pltpu.make_async_copy(
                w_hbm.at[idx_v.at[c]], rows_v.at[b], sems.at[b]
            ).wait()
            for s in range(_NB):
                pltpu.sync_copy(
                    rows_v.at[b].at[pl.ds(s * h, h)],
                    out_hbm.at[batch0 + c * _NB + s],
                )

            @pl.when(c + _NBUF < n_chunks)
            def _():
                pltpu.async_copy(
                    w_hbm.at[idx_v.at[c + _NBUF]], rows_v.at[b], sems.at[b]
                )

        return carry

    lax.fori_loop(0, n_chunks // _NBUF, group, 0)


@functools.partial(jax.jit, static_argnums=(2, 3, 4))
def _embedding_lookup(idx, w, n_chunks, h, d):
    mesh = plsc.VectorSubcoreMesh(core_axis_name="c", subcore_axis_name="s")
    out = pl.kernel(
        functools.partial(_gather_body, n_chunks, h),
        out_type=jax.ShapeDtypeStruct((_NW * n_chunks * _NB, h, d), w.dtype),
        mesh=mesh,
        scratch_types=[
            pltpu.VMEM((n_chunks, _NB * h), jnp.int32),
            pltpu.VMEM((_NBUF, _NB * h, d), w.dtype),
            pltpu.SemaphoreType.DMA((_NBUF,)),
        ],
    )(idx, w)
    return out


def kernel(token_ids, W):
    b, h = token_ids.shape
    n, d = W.shape
    assert b % (_NSPLIT * _NW * _NB) == 0
    bs = b // _NSPLIT
    n_chunks = bs // (_NW * _NB)
    assert n_chunks % _NBUF == 0
    ids = token_ids.astype(jnp.int32)
    parts = []
    for i in range(_NSPLIT):
        idx = ids[i * bs:(i + 1) * bs].reshape(_NW, n_chunks, _NB * h)
        parts.append(_embedding_lookup(idx, W, n_chunks, h, d))
    out = jnp.zeros((b, h, d), W.dtype)
    for i in range(_NSPLIT):
        out = lax.dynamic_update_slice(out, parts[i], (i * bs, 0, 0))
    return out


# speedup vs baseline: 1.6892x; 1.6892x over previous
import functools

import jax
import jax.numpy as jnp
from jax import lax
from jax.experimental import pallas as pl
from jax.experimental.pallas import tpu as pltpu
from jax.experimental.pallas import tpu_sc as plsc

_NUM_CORES = 2
_NUM_SUBCORES = 16
_NW = _NUM_CORES * _NUM_SUBCORES
_NB = 2
_NBUF = 8
_NSPLIT = 1


def _gather_body(n_chunks, h, idx_hbm, w_hbm, out_hbm, idx_v, rows_v, gsems,
                 wsems):
    wid = lax.axis_index("s") * _NUM_CORES + lax.axis_index("c")
    pltpu.sync_copy(idx_hbm.at[wid], idx_v)
    batch0 = wid * (n_chunks * _NB)
    n_groups = n_chunks // _NBUF

    def _wait_writeback(br, cw):
        for s in range(_NB):
            pltpu.make_async_copy(
                rows_v.at[br].at[pl.ds(s * h, h)],
                out_hbm.at[batch0 + cw * _NB + s],
                wsems.at[br],
            ).wait()

    for b in range(_NBUF - 2):
        pltpu.async_copy(w_hbm.at[idx_v.at[b]], rows_v.at[b], gsems.at[b])

    def group(g, carry):
        for b in range(_NBUF):
            c = g * _NBUF + b
            pltpu.make_async_copy(
                w_hbm.at[idx_v.at[c]], rows_v.at[b], gsems.at[b]
            ).wait()
            for s in range(_NB):
                pltpu.async_copy(
                    rows_v.at[b].at[pl.ds(s * h, h)],
                    out_hbm.at[batch0 + c * _NB + s],
                    wsems.at[b],
                )

            br = (b - 2) % _NBUF
            if b >= 2:
                _wait_writeback(br, c - 2)
                @pl.when(g < n_groups - 1)
                def _():
                    pltpu.async_copy(
                        w_hbm.at[idx_v.at[c + _NBUF - 2]],
                        rows_v.at[br],
                        gsems.at[br],
                    )
            else:
                @pl.when(g > 0)
                def _():
                    _wait_writeback(br, c - 2)
                pltpu.async_copy(
                    w_hbm.at[idx_v.at[c + _NBUF - 2]],
                    rows_v.at[br],
                    gsems.at[br],
                )

        return carry

    lax.fori_loop(0, n_groups, group, 0)
    for cw in (n_chunks - 2, n_chunks - 1):
        _wait_writeback(cw % _NBUF, cw)


@functools.partial(jax.jit, static_argnums=(2, 3, 4))
def _embedding_lookup(idx, w, n_chunks, h, d):
    mesh = plsc.VectorSubcoreMesh(core_axis_name="c", subcore_axis_name="s")
    out = pl.kernel(
        functools.partial(_gather_body, n_chunks, h),
        out_type=jax.ShapeDtypeStruct((_NW * n_chunks * _NB, h, d), w.dtype),
        mesh=mesh,
        scratch_types=[
            pltpu.VMEM((n_chunks, _NB * h), jnp.int32),
            pltpu.VMEM((_NBUF, _NB * h, d), w.dtype),
            pltpu.SemaphoreType.DMA((_NBUF,)),
            pltpu.SemaphoreType.DMA((_NBUF,)),
        ],
    )(idx, w)
    return out


def kernel(token_ids, W):
    b, h = token_ids.shape
    n, d = W.shape
    assert b % (_NSPLIT * _NW * _NB) == 0
    bs = b // _NSPLIT
    n_chunks = bs // (_NW * _NB)
    assert n_chunks % _NBUF == 0
    ids = token_ids.astype(jnp.int32)
    parts = []
    for i in range(_NSPLIT):
        idx = ids[i * bs:(i + 1) * bs].reshape(_NW, n_chunks, _NB * h)
        parts.append(_embedding_lookup(idx, W, n_chunks, h, d))
    if _NSPLIT == 1:
        return parts[0]
    return jnp.concatenate(parts, axis=0)
